# baseline (speedup 1.0000x reference)
import jax
import jax.numpy as jnp
from jax import lax
from jax.experimental import pallas as pl
from jax.experimental.pallas import tpu as pltpu
from jax.experimental.pallas import tpu_sc as plsc

T = 1000
D = 128
B = 16384

_info = plsc.get_sparse_core_info()
_NC, _NS = _info.num_cores, _info.num_subcores
_NW = _NC * _NS
_BPW = B // _NW
_NCH = 2
_CHUNK = _BPW // _NCH


def _gather_kernel(table_hbm, t_hbm, out_hbm, idx_v, rows_v, gs0, gs1, ws0, ws1):
    wid = lax.axis_index("s") * _NC + lax.axis_index("c")
    base = wid * _BPW
    pltpu.sync_copy(t_hbm.at[pl.ds(base, _BPW)], idx_v)
    g0 = pltpu.async_copy(
        table_hbm.at[idx_v.at[pl.ds(0, _CHUNK)]], rows_v.at[0], gs0)
    g1 = pltpu.async_copy(
        table_hbm.at[idx_v.at[pl.ds(_CHUNK, _CHUNK)]], rows_v.at[1], gs1)
    g0.wait()
    w0 = pltpu.async_copy(rows_v.at[0], out_hbm.at[pl.ds(base, _CHUNK)], ws0)
    g1.wait()
    w1 = pltpu.async_copy(
        rows_v.at[1], out_hbm.at[pl.ds(base + _CHUNK, _CHUNK)], ws1)
    w0.wait()
    w1.wait()


@jax.jit
def _lookup(table, t):
    mesh = plsc.VectorSubcoreMesh(core_axis_name="c", subcore_axis_name="s")
    return pl.kernel(
        _gather_kernel,
        mesh=mesh,
        out_type=jax.ShapeDtypeStruct((B, D), jnp.float32),
        scratch_types=[
            pltpu.VMEM((_BPW,), jnp.int32),
            pltpu.VMEM((2, _CHUNK, D), jnp.float32),
            pltpu.SemaphoreType.DMA,
            pltpu.SemaphoreType.DMA,
            pltpu.SemaphoreType.DMA,
            pltpu.SemaphoreType.DMA,
        ],
    )(table, t)


def kernel(table, t):
    return _lookup(table, t.astype(jnp.int32))


# speedup vs baseline: 1.0600x; 1.0600x over previous
import jax
import jax.numpy as jnp
from jax import lax
from jax.experimental import pallas as pl
from jax.experimental.pallas import tpu as pltpu
from jax.experimental.pallas import tpu_sc as plsc

T = 1000
D = 128
B = 16384

_info = plsc.get_sparse_core_info()
_NC, _NS = _info.num_cores, _info.num_subcores
_NW = _NC * _NS
_BPW = B // _NW
_NCH = 2
_CHUNK = _BPW // _NCH


def _gather_kernel(table_hbm, t_hbm, out_hbm, idx_v, rows_v, sem):
    wid = lax.axis_index("s") * _NC + lax.axis_index("c")
    base = wid * _BPW
    pltpu.sync_copy(t_hbm.at[pl.ds(base, _BPW)], idx_v)
    pltpu.async_copy(table_hbm.at[idx_v], rows_v, sem).wait()
    pltpu.sync_copy(rows_v, out_hbm.at[pl.ds(base, _BPW)])


@jax.jit
def _lookup(table, t):
    mesh = plsc.VectorSubcoreMesh(core_axis_name="c", subcore_axis_name="s")
    return pl.kernel(
        _gather_kernel,
        mesh=mesh,
        out_type=jax.ShapeDtypeStruct((B, D), jnp.float32),
        scratch_types=[
            pltpu.VMEM((_BPW,), jnp.int32),
            pltpu.VMEM((_BPW, D), jnp.float32),
            pltpu.SemaphoreType.DMA,
        ],
    )(table, t)


def kernel(table, t):
    return _lookup(table, t.astype(jnp.int32))


# speedup vs baseline: 1.1549x; 1.0895x over previous
import jax
import jax.numpy as jnp
from jax import lax
from jax.experimental import pallas as pl
from jax.experimental.pallas import tpu as pltpu
from jax.experimental.pallas import tpu_sc as plsc

T = 1000
D = 128
B = 16384

_info = plsc.get_sparse_core_info()
_NC, _NS = _info.num_cores, _info.num_subcores
_NW = _NC * _NS
_BPW = B // _NW
_NCH = 2
_CHUNK = _BPW // _NCH


_TROWS = 64


def _gather_kernel(table_hbm, t_hbm, out_hbm, table_sp, idx_v, rows_v, sem):
    sid = lax.axis_index("s")
    wid = sid * _NC + lax.axis_index("c")
    base = wid * _BPW
    pltpu.sync_copy(t_hbm.at[pl.ds(base, _BPW)], idx_v)

    @pl.when(sid < 15)
    def _stage_table():
        r0 = sid * _TROWS
        pltpu.sync_copy(table_hbm.at[pl.ds(r0, _TROWS)],
                        table_sp.at[pl.ds(r0, _TROWS)])

    @pl.when(sid == 15)
    def _stage_tail():
        pltpu.sync_copy(table_hbm.at[pl.ds(15 * _TROWS, T - 15 * _TROWS)],
                        table_sp.at[pl.ds(15 * _TROWS, T - 15 * _TROWS)])

    plsc.subcore_barrier()
    pltpu.async_copy(table_sp.at[idx_v], rows_v, sem).wait()
    pltpu.sync_copy(rows_v, out_hbm.at[pl.ds(base, _BPW)])


@jax.jit
def _lookup(table, t):
    mesh = plsc.VectorSubcoreMesh(core_axis_name="c", subcore_axis_name="s")
    return pl.kernel(
        _gather_kernel,
        mesh=mesh,
        out_type=jax.ShapeDtypeStruct((B, D), jnp.float32),
        scratch_types=[
            pltpu.VMEM_SHARED((T, D), jnp.float32),
            pltpu.VMEM((_BPW,), jnp.int32),
            pltpu.VMEM((_BPW, D), jnp.float32),
            pltpu.SemaphoreType.DMA,
        ],
    )(table, t)


def kernel(table, t):
    return _lookup(table, t.astype(jnp.int32))


# speedup vs baseline: 1.1666x; 1.0102x over previous
import jax
import jax.numpy as jnp
from jax import lax
from jax.experimental import pallas as pl
from jax.experimental.pallas import tpu as pltpu
from jax.experimental.pallas import tpu_sc as plsc

T = 1000
D = 128
B = 16384

_info = plsc.get_sparse_core_info()
_NC, _NS = _info.num_cores, _info.num_subcores
_NW = _NC * _NS
_BPW = B // _NW
_NCH = 2
_CHUNK = _BPW // _NCH


_TROWS = 64


def _gather_kernel(table_hbm, t_hbm, out_hbm, table_sp, idx_v, rows_v,
                   si, gs0, gs1, ws0, ws1):
    sid = lax.axis_index("s")
    wid = sid * _NC + lax.axis_index("c")
    base = wid * _BPW
    idx_cp = pltpu.async_copy(t_hbm.at[pl.ds(base, _BPW)], idx_v, si)

    @pl.when(sid < 15)
    def _stage_table():
        r0 = sid * _TROWS
        pltpu.sync_copy(table_hbm.at[pl.ds(r0, _TROWS)],
                        table_sp.at[pl.ds(r0, _TROWS)])

    @pl.when(sid == 15)
    def _stage_tail():
        pltpu.sync_copy(table_hbm.at[pl.ds(15 * _TROWS, T - 15 * _TROWS)],
                        table_sp.at[pl.ds(15 * _TROWS, T - 15 * _TROWS)])

    plsc.subcore_barrier()
    idx_cp.wait()
    g0 = pltpu.async_copy(
        table_sp.at[idx_v.at[pl.ds(0, _CHUNK)]], rows_v.at[0], gs0)
    g1 = pltpu.async_copy(
        table_sp.at[idx_v.at[pl.ds(_CHUNK, _CHUNK)]], rows_v.at[1], gs1)
    g0.wait()
    w0 = pltpu.async_copy(rows_v.at[0], out_hbm.at[pl.ds(base, _CHUNK)], ws0)
    g1.wait()
    w1 = pltpu.async_copy(
        rows_v.at[1], out_hbm.at[pl.ds(base + _CHUNK, _CHUNK)], ws1)
    w0.wait()
    w1.wait()


@jax.jit
def _lookup(table, t):
    mesh = plsc.VectorSubcoreMesh(core_axis_name="c", subcore_axis_name="s")
    return pl.kernel(
        _gather_kernel,
        mesh=mesh,
        out_type=jax.ShapeDtypeStruct((B, D), jnp.float32),
        scratch_types=[
            pltpu.VMEM_SHARED((T, D), jnp.float32),
            pltpu.VMEM((_BPW,), jnp.int32),
            pltpu.VMEM((2, _CHUNK, D), jnp.float32),
            pltpu.SemaphoreType.DMA,
            pltpu.SemaphoreType.DMA,
            pltpu.SemaphoreType.DMA,
            pltpu.SemaphoreType.DMA,
            pltpu.SemaphoreType.DMA,
        ],
    )(table, t)


def kernel(table, t):
    return _lookup(table, t.astype(jnp.int32))


# speedup vs baseline: 1.1968x; 1.0258x over previous
import jax
import jax.numpy as jnp
from jax import lax
from jax.experimental import pallas as pl
from jax.experimental.pallas import tpu as pltpu
from jax.experimental.pallas import tpu_sc as plsc

T = 1000
D = 128
B = 16384

_info = plsc.get_sparse_core_info()
_NC, _NS = _info.num_cores, _info.num_subcores
_NW = _NC * _NS
_BPW = B // _NW
_NCH = 4
_CHUNK = _BPW // _NCH


_TROWS = 64


def _gather_kernel(table_hbm, t_hbm, out_hbm, table_sp, idx_v, rows_v,
                   si, gsem, wsem):
    sid = lax.axis_index("s")
    wid = sid * _NC + lax.axis_index("c")
    base = wid * _BPW
    idx_cp = pltpu.async_copy(t_hbm.at[pl.ds(base, _BPW)], idx_v, si)

    @pl.when(sid < 15)
    def _stage_table():
        r0 = sid * _TROWS
        pltpu.sync_copy(table_hbm.at[pl.ds(r0, _TROWS)],
                        table_sp.at[pl.ds(r0, _TROWS)])

    @pl.when(sid == 15)
    def _stage_tail():
        pltpu.sync_copy(table_hbm.at[pl.ds(15 * _TROWS, T - 15 * _TROWS)],
                        table_sp.at[pl.ds(15 * _TROWS, T - 15 * _TROWS)])

    plsc.subcore_barrier()
    idx_cp.wait()
    gathers = []
    for i in range(_NCH):
        gathers.append(pltpu.async_copy(
            table_sp.at[idx_v.at[pl.ds(i * _CHUNK, _CHUNK)]],
            rows_v.at[i], gsem.at[i]))
    writes = []
    for i in range(_NCH):
        gathers[i].wait()
        writes.append(pltpu.async_copy(
            rows_v.at[i], out_hbm.at[pl.ds(base + i * _CHUNK, _CHUNK)],
            wsem.at[i]))
    for w in writes:
        w.wait()


@jax.jit
def _lookup(table, t):
    mesh = plsc.VectorSubcoreMesh(core_axis_name="c", subcore_axis_name="s")
    return pl.kernel(
        _gather_kernel,
        mesh=mesh,
        out_type=jax.ShapeDtypeStruct((B, D), jnp.float32),
        scratch_types=[
            pltpu.VMEM_SHARED((T, D), jnp.float32),
            pltpu.VMEM((_BPW,), jnp.int32),
            pltpu.VMEM((_NCH, _CHUNK, D), jnp.float32),
            pltpu.SemaphoreType.DMA,
            pltpu.SemaphoreType.DMA((_NCH,)),
            pltpu.SemaphoreType.DMA((_NCH,)),
        ],
    )(table, t)


def kernel(table, t):
    return _lookup(table, t.astype(jnp.int32))


# speedup vs baseline: 1.2136x; 1.0140x over previous
import jax
import jax.numpy as jnp
from jax import lax
from jax.experimental import pallas as pl
from jax.experimental.pallas import tpu as pltpu
from jax.experimental.pallas import tpu_sc as plsc

T = 1000
D = 128
B = 16384

_info = plsc.get_sparse_core_info()
_NC, _NS = _info.num_cores, _info.num_subcores
_NW = _NC * _NS
_BPW = B // _NW
_NCH = 8
_CHUNK = _BPW // _NCH


_TROWS = 64


def _gather_kernel(table_hbm, t_hbm, out_hbm, table_sp, idx_v, rows_v,
                   si, gsem, wsem):
    sid = lax.axis_index("s")
    wid = sid * _NC + lax.axis_index("c")
    base = wid * _BPW
    idx_cp = pltpu.async_copy(t_hbm.at[pl.ds(base, _BPW)], idx_v, si)

    @pl.when(sid < 15)
    def _stage_table():
        r0 = sid * _TROWS
        pltpu.sync_copy(table_hbm.at[pl.ds(r0, _TROWS)],
                        table_sp.at[pl.ds(r0, _TROWS)])

    @pl.when(sid == 15)
    def _stage_tail():
        pltpu.sync_copy(table_hbm.at[pl.ds(15 * _TROWS, T - 15 * _TROWS)],
                        table_sp.at[pl.ds(15 * _TROWS, T - 15 * _TROWS)])

    plsc.subcore_barrier()
    idx_cp.wait()
    gathers = []
    for i in range(_NCH):
        gathers.append(pltpu.async_copy(
            table_sp.at[idx_v.at[pl.ds(i * _CHUNK, _CHUNK)]],
            rows_v.at[i], gsem.at[i]))
    writes = []
    for i in range(_NCH):
        gathers[i].wait()
        writes.append(pltpu.async_copy(
            rows_v.at[i], out_hbm.at[pl.ds(base + i * _CHUNK, _CHUNK)],
            wsem.at[i]))
    for w in writes:
        w.wait()


@jax.jit
def _lookup(table, t):
    mesh = plsc.VectorSubcoreMesh(core_axis_name="c", subcore_axis_name="s")
    return pl.kernel(
        _gather_kernel,
        mesh=mesh,
        out_type=jax.ShapeDtypeStruct((B, D), jnp.float32),
        scratch_types=[
            pltpu.VMEM_SHARED((T, D), jnp.float32),
            pltpu.VMEM((_BPW,), jnp.int32),
            pltpu.VMEM((_NCH, _CHUNK, D), jnp.float32),
            pltpu.SemaphoreType.DMA,
            pltpu.SemaphoreType.DMA((_NCH,)),
            pltpu.SemaphoreType.DMA((_NCH,)),
        ],
    )(table, t)


def kernel(table, t):
    return _lookup(table, t.astype(jnp.int32))
